# Initial kernel scaffold; baseline (speedup 1.0000x reference)
import functools

import jax
import jax.numpy as jnp
from jax import lax
from jax.experimental import pallas as pl
from jax.experimental.pallas import tpu as pltpu
from jax.experimental.pallas import tpu_sc as plsc

N_CLUSTERS = 100000
HIDDEN = 64
SEQ = 200

NUM_CORES = 2
NUM_SUBCORES = 16
NW = NUM_CORES * NUM_SUBCORES

CHUNK = 2 * SEQ
GSLICE = 80
NGS = CHUNK // GSLICE
TOTAL = 4096 * SEQ
NCHUNKS = TOTAL // CHUNK
CH_PER_W = NCHUNKS // NW


def _body(x_hbm, pix_hbm, pos_hbm, out_hbm, idx_v, rows_v, pos_v, sem):
    wid = lax.axis_index("s") * NUM_CORES + lax.axis_index("c")

    pltpu.sync_copy(pos_hbm.at[pl.ds(0, SEQ)], pos_v)

    def chunk_body(c, _):
        row = wid * CH_PER_W + c
        pltpu.sync_copy(x_hbm.at[row], idx_v)
        for k in range(NGS):
            pltpu.async_copy(
                pix_hbm.at[idx_v.at[k]],
                rows_v.at[pl.ds(k * GSLICE, GSLICE)],
                sem,
            ).wait()

        def add_body(r, _):
            for j in range(HIDDEN // 16):
                sl = pl.ds(j * 16, 16)
                p = pos_v[r, sl]
                plsc.addupdate(rows_v.at[r, sl], p)
                plsc.addupdate(rows_v.at[SEQ + r, sl], p)
            return 0

        lax.fori_loop(0, SEQ, add_body, 0)
        pltpu.sync_copy(rows_v, out_hbm.at[row])
        return 0

    lax.fori_loop(0, CH_PER_W, chunk_body, 0)


def kernel(x, pix_table, pos_table):
    b, seq = x.shape
    x3 = x.astype(jnp.int32).reshape(NCHUNKS, NGS, GSLICE)
    mesh = plsc.VectorSubcoreMesh(core_axis_name="c", subcore_axis_name="s")
    run = functools.partial(
        pl.kernel,
        mesh=mesh,
        out_type=jax.ShapeDtypeStruct((NCHUNKS, CHUNK, HIDDEN), jnp.float32),
        scratch_types=[
            pltpu.VMEM((NGS, GSLICE), jnp.int32),
            pltpu.VMEM((CHUNK, HIDDEN), jnp.float32),
            pltpu.VMEM((SEQ, HIDDEN), jnp.float32),
            pltpu.SemaphoreType.DMA,
        ],
    )(_body)
    out = run(x3, pix_table, pos_table)
    return out.reshape(b, seq, HIDDEN)


# speedup vs baseline: 8.0344x; 8.0344x over previous
import functools

import jax
import jax.numpy as jnp
from jax import lax
from jax.experimental import pallas as pl
from jax.experimental.pallas import tpu as pltpu
from jax.experimental.pallas import tpu_sc as plsc

N_CLUSTERS = 100000
HIDDEN = 64
SEQ = 200

NUM_CORES = 2
NUM_SUBCORES = 16
NW = NUM_CORES * NUM_SUBCORES

CHUNK = 2 * SEQ
GSLICE = 80
NGS = CHUNK // GSLICE
TOTAL = 4096 * SEQ
NCHUNKS = TOTAL // CHUNK
CH_PER_W = NCHUNKS // NW


def _body(x_hbm, pix_hbm, pos_hbm, out_hbm,
          idx_all, rows0, rows1, pos_v, sem_g0, sem_g1, sem_o0, sem_o1):
    wid = lax.axis_index("s") * NUM_CORES + lax.axis_index("c")
    base = wid * CH_PER_W
    rows = (rows0, rows1)
    sem_g = (sem_g0, sem_g1)
    sem_o = (sem_o0, sem_o1)

    pltpu.sync_copy(pos_hbm.at[pl.ds(0, SEQ)], pos_v)
    pltpu.sync_copy(x_hbm.at[wid], idx_all)

    def fire_gathers(c, b):
        for k in range(NGS):
            pltpu.async_copy(
                pix_hbm.at[idx_all.at[c, k]],
                rows[b].at[pl.ds(k * GSLICE, GSLICE)],
                sem_g[b],
            )

    fire_gathers(0, 0)

    @pl.loop(0, CH_PER_W, step=2)
    def _chunk_pair(c2):
        for b in range(2):
            c = c2 + b
            nb = 1 - b

            @pl.when(c + 1 < CH_PER_W)
            def _fire_next():
                @pl.when(c > 0)
                def _drain_prev_out():
                    pltpu.make_async_copy(
                        rows[nb], out_hbm.at[base + c - 1], sem_o[nb]
                    ).wait()
                fire_gathers(c + 1, nb)

            pltpu.make_async_copy(
                pix_hbm.at[pl.ds(0, CHUNK)], rows[b], sem_g[b]
            ).wait()

            @pl.loop(0, SEQ, unroll=8)
            def _add(r):
                for j in range(HIDDEN // 16):
                    sl = pl.ds(j * 16, 16)
                    p = pos_v[r, sl]
                    plsc.addupdate(rows[b].at[r, sl], p)
                    plsc.addupdate(rows[b].at[SEQ + r, sl], p)

            pltpu.async_copy(rows[b], out_hbm.at[base + c], sem_o[b])

    lastb = (CH_PER_W - 1) % 2
    pltpu.make_async_copy(
        rows[1 - lastb], out_hbm.at[base + CH_PER_W - 2], sem_o[1 - lastb]
    ).wait()
    pltpu.make_async_copy(
        rows[lastb], out_hbm.at[base + CH_PER_W - 1], sem_o[lastb]
    ).wait()


def kernel(x, pix_table, pos_table):
    b, seq = x.shape
    x4 = x.astype(jnp.int32).reshape(NW, CH_PER_W, NGS, GSLICE)
    mesh = plsc.VectorSubcoreMesh(core_axis_name="c", subcore_axis_name="s")
    run = functools.partial(
        pl.kernel,
        mesh=mesh,
        out_type=jax.ShapeDtypeStruct((NCHUNKS, CHUNK, HIDDEN), jnp.float32),
        scratch_types=[
            pltpu.VMEM((CH_PER_W, NGS, GSLICE), jnp.int32),
            pltpu.VMEM((CHUNK, HIDDEN), jnp.float32),
            pltpu.VMEM((CHUNK, HIDDEN), jnp.float32),
            pltpu.VMEM((SEQ, HIDDEN), jnp.float32),
            pltpu.SemaphoreType.DMA,
            pltpu.SemaphoreType.DMA,
            pltpu.SemaphoreType.DMA,
            pltpu.SemaphoreType.DMA,
        ],
        compiler_params=pltpu.CompilerParams(use_tc_tiling_on_sc=False),
    )(_body)
    out = run(x4, pix_table, pos_table)
    return out.reshape(b, seq, HIDDEN)


# speedup vs baseline: 8.0453x; 1.0014x over previous
import functools

import jax
import jax.numpy as jnp
from jax import lax
from jax.experimental import pallas as pl
from jax.experimental.pallas import tpu as pltpu
from jax.experimental.pallas import tpu_sc as plsc

N_CLUSTERS = 100000
HIDDEN = 64
SEQ = 200

NUM_CORES = 2
NUM_SUBCORES = 16
NW = NUM_CORES * NUM_SUBCORES

CHUNK = 2 * SEQ
GSLICE = 80
NGS = CHUNK // GSLICE
TOTAL = 4096 * SEQ
NCHUNKS = TOTAL // CHUNK
CH_PER_W = NCHUNKS // NW


def _body(x_hbm, pix_hbm, pos_hbm, out_hbm,
          idx_all, rows0, rows1, pos_v, sem_g0, sem_g1, sem_o0, sem_o1):
    wid = lax.axis_index("s") * NUM_CORES + lax.axis_index("c")
    base2 = wid * (2 * CH_PER_W)
    rows = (rows0, rows1)
    sem_g = (sem_g0, sem_g1)
    sem_o = (sem_o0, sem_o1)

    pltpu.sync_copy(pos_hbm.at[pl.ds(0, SEQ)], pos_v)
    pltpu.sync_copy(x_hbm.at[wid], idx_all)

    def fire_gathers(c, b):
        for k in range(NGS):
            pltpu.async_copy(
                pix_hbm.at[idx_all.at[c, k]],
                rows[b].at[pl.ds(k * GSLICE, GSLICE)],
                sem_g[b],
            )

    def fire_out(c, b):
        pltpu.async_copy(rows[b].at[pl.ds(0, SEQ)], out_hbm.at[base2 + 2 * c], sem_o[b])
        pltpu.async_copy(rows[b].at[pl.ds(SEQ, SEQ)], out_hbm.at[base2 + 2 * c + 1], sem_o[b])

    def drain_out(c, b):
        pltpu.make_async_copy(
            rows[b].at[pl.ds(0, SEQ)], out_hbm.at[base2 + 2 * c], sem_o[b]
        ).wait()
        pltpu.make_async_copy(
            rows[b].at[pl.ds(SEQ, SEQ)], out_hbm.at[base2 + 2 * c + 1], sem_o[b]
        ).wait()

    fire_gathers(0, 0)

    @pl.loop(0, CH_PER_W, step=2)
    def _chunk_pair(c2):
        for b in range(2):
            c = c2 + b
            nb = 1 - b

            @pl.when(c + 1 < CH_PER_W)
            def _fire_next():
                @pl.when(c > 0)
                def _drain_prev_out():
                    drain_out(c - 1, nb)
                fire_gathers(c + 1, nb)

            pltpu.make_async_copy(
                pix_hbm.at[pl.ds(0, CHUNK)], rows[b], sem_g[b]
            ).wait()

            @pl.loop(0, SEQ, unroll=8)
            def _add(r):
                for j in range(HIDDEN // 16):
                    sl = pl.ds(j * 16, 16)
                    p = pos_v[r, sl]
                    plsc.addupdate(rows[b].at[r, sl], p)
                    plsc.addupdate(rows[b].at[SEQ + r, sl], p)

            fire_out(c, b)

    lastb = (CH_PER_W - 1) % 2
    drain_out(CH_PER_W - 2, 1 - lastb)
    drain_out(CH_PER_W - 1, lastb)


def kernel(x, pix_table, pos_table):
    b, seq = x.shape
    x4 = x.astype(jnp.int32).reshape(NW, CH_PER_W, NGS, GSLICE)
    mesh = plsc.VectorSubcoreMesh(core_axis_name="c", subcore_axis_name="s")
    run = functools.partial(
        pl.kernel,
        mesh=mesh,
        out_type=jax.ShapeDtypeStruct((4096, SEQ, HIDDEN), jnp.float32),
        scratch_types=[
            pltpu.VMEM((CH_PER_W, NGS, GSLICE), jnp.int32),
            pltpu.VMEM((CHUNK, HIDDEN), jnp.float32),
            pltpu.VMEM((CHUNK, HIDDEN), jnp.float32),
            pltpu.VMEM((SEQ, HIDDEN), jnp.float32),
            pltpu.SemaphoreType.DMA,
            pltpu.SemaphoreType.DMA,
            pltpu.SemaphoreType.DMA,
            pltpu.SemaphoreType.DMA,
        ],
        compiler_params=pltpu.CompilerParams(use_tc_tiling_on_sc=False),
    )(_body)
    return run(x4, pix_table, pos_table)


# speedup vs baseline: 8.1328x; 1.0109x over previous
import functools

import jax
import jax.numpy as jnp
from jax import lax
from jax.experimental import pallas as pl
from jax.experimental.pallas import tpu as pltpu
from jax.experimental.pallas import tpu_sc as plsc

N_CLUSTERS = 100000
HIDDEN = 64
SEQ = 200

NUM_CORES = 2
NUM_SUBCORES = 16
NW = NUM_CORES * NUM_SUBCORES

CHUNK = 2 * SEQ
GSLICE = 80
NGS = CHUNK // GSLICE
TOTAL = 4096 * SEQ
NCHUNKS = TOTAL // CHUNK
CH_PER_W = NCHUNKS // NW
NBUF = 3
LOOK = 1


def _body(x_hbm, pix_hbm, pos_hbm, out_hbm,
          idx_all, rows0, rows1, rows2, pos_v,
          sem_g0, sem_g1, sem_g2,
          sem_o0, sem_o1, sem_o2):
    wid = lax.axis_index("s") * NUM_CORES + lax.axis_index("c")
    base2 = wid * (2 * CH_PER_W)
    rows = (rows0, rows1, rows2)
    sem_g = (sem_g0, sem_g1, sem_g2)
    sem_o = (sem_o0, sem_o1, sem_o2)

    pltpu.sync_copy(pos_hbm.at[pl.ds(0, SEQ)], pos_v)
    pltpu.sync_copy(x_hbm.at[wid], idx_all)

    def fire_gathers(c, b):
        for k in range(NGS):
            pltpu.async_copy(
                pix_hbm.at[idx_all.at[c, k]],
                rows[b].at[pl.ds(k * GSLICE, GSLICE)],
                sem_g[b],
            )

    def fire_out(c, b):
        pltpu.async_copy(rows[b].at[pl.ds(0, SEQ)], out_hbm.at[base2 + 2 * c], sem_o[b])
        pltpu.async_copy(rows[b].at[pl.ds(SEQ, SEQ)], out_hbm.at[base2 + 2 * c + 1], sem_o[b])

    def drain_out(c, b):
        pltpu.make_async_copy(
            rows[b].at[pl.ds(0, SEQ)], out_hbm.at[base2 + 2 * c], sem_o[b]
        ).wait()
        pltpu.make_async_copy(
            rows[b].at[pl.ds(SEQ, SEQ)], out_hbm.at[base2 + 2 * c + 1], sem_o[b]
        ).wait()

    def _chunk_step(c, b, bn):
            @pl.when(c + LOOK < CH_PER_W)
            def _fire_ahead():
                @pl.when(c + LOOK >= NBUF)
                def _drain_prev_out():
                    drain_out(c + LOOK - NBUF, bn)
                fire_gathers(c + LOOK, bn)

            pltpu.make_async_copy(
                pix_hbm.at[pl.ds(0, CHUNK)], rows[b], sem_g[b]
            ).wait()

            @pl.loop(0, SEQ, unroll=8)
            def _add(r):
                for j in range(HIDDEN // 16):
                    sl = pl.ds(j * 16, 16)
                    p = pos_v[r, sl]
                    plsc.addupdate(rows[b].at[r, sl], p)
                    plsc.addupdate(rows[b].at[SEQ + r, sl], p)

            fire_out(c, b)

    for p in range(LOOK):
        fire_gathers(p, p)

    @pl.loop(0, CH_PER_W, step=NBUF)
    def _chunk_grp(c0):
        for b in range(NBUF):
            c = c0 + b
            bn = (b + LOOK) % NBUF

            @pl.when(c < CH_PER_W)
            def _in_range():
                _chunk_step(c, b, bn)

    for q in range(NBUF):
        c = CH_PER_W - NBUF + q
        drain_out(c, c % NBUF)


def kernel(x, pix_table, pos_table):
    b, seq = x.shape
    x4 = x.astype(jnp.int32).reshape(NW, CH_PER_W, NGS, GSLICE)
    mesh = plsc.VectorSubcoreMesh(core_axis_name="c", subcore_axis_name="s")
    run = functools.partial(
        pl.kernel,
        mesh=mesh,
        out_type=jax.ShapeDtypeStruct((4096, SEQ, HIDDEN), jnp.float32),
        scratch_types=[
            pltpu.VMEM((CH_PER_W, NGS, GSLICE), jnp.int32),
            pltpu.VMEM((CHUNK, HIDDEN), jnp.float32),
            pltpu.VMEM((CHUNK, HIDDEN), jnp.float32),
            pltpu.VMEM((CHUNK, HIDDEN), jnp.float32),
            pltpu.VMEM((SEQ, HIDDEN), jnp.float32),
            pltpu.SemaphoreType.DMA,
            pltpu.SemaphoreType.DMA,
            pltpu.SemaphoreType.DMA,
            pltpu.SemaphoreType.DMA,
            pltpu.SemaphoreType.DMA,
            pltpu.SemaphoreType.DMA,
        ],
        compiler_params=pltpu.CompilerParams(use_tc_tiling_on_sc=False),
    )(_body)
    return run(x4, pix_table, pos_table)
